# Initial kernel scaffold; baseline (speedup 1.0000x reference)
import jax
import jax.numpy as jnp
from jax.experimental import pallas as pl


def kernel(node_features, edge_index, edge_features, ew1, eb1, ew2, eb2, nw1, nb1, nw2, nb2, eg, ebn, ng, nbn):
    raise NotImplementedError("write your pallas kernel here")


# speedup vs baseline: 2.9467x; 2.9467x over previous
import functools

import jax
import jax.numpy as jnp
from jax import lax
from jax.experimental import pallas as pl
from jax.experimental.pallas import tpu as pltpu
from jax.experimental.pallas import tpu_sc as plsc


def _pre_body(nf_ref, ew1_ref, ps_ref, pd_ref):
    nf = nf_ref[...]
    nd = nf.shape[1]
    ws = ew1_ref[pl.ds(0, nd), :]
    wd = ew1_ref[pl.ds(nd, nd), :]
    ps_ref[...] = jnp.dot(nf, ws, preferred_element_type=jnp.float32)
    pd_ref[...] = jnp.dot(nf, wd, preferred_element_type=jnp.float32)


def _gelu(x):
    return 0.5 * x * (1.0 + lax.erf(x * 0.7071067811865476))


def _edge_body(gs_ref, gd_ref, ef_ref, ew1_ref, eb1_ref, ew2_ref, eb2_ref,
               eg_ref, ebn_ref, eo_ref):
    ef = ef_ref[...]
    ed = ef.shape[1]
    nd2 = ew1_ref.shape[0] - ed
    we = ew1_ref[pl.ds(nd2, ed), :]
    x = gs_ref[...] + gd_ref[...]
    x = x + jnp.dot(ef, we, preferred_element_type=jnp.float32) + eb1_ref[...]
    h = _gelu(x)
    upd = jnp.dot(h, ew2_ref[...], preferred_element_type=jnp.float32)
    eo = ef + upd + eb2_ref[...]
    mu = jnp.mean(eo, axis=-1, keepdims=True)
    var = jnp.mean((eo - mu) ** 2, axis=-1, keepdims=True)
    eo_ref[...] = (eo - mu) * lax.rsqrt(var + 1e-5) * eg_ref[...] + ebn_ref[...]


def _node_body(nf_ref, agg_ref, deg_ref, nw1_ref,
               nb1_ref, nw2_ref, nb2_ref, ng_ref, nbn_ref, no_ref):
    nf = nf_ref[...]
    nd = nf.shape[1]
    cnt = jnp.maximum(deg_ref[:, 0:1], 1.0)
    agg = agg_ref[...] / cnt
    w_nf = nw1_ref[pl.ds(0, nd), :]
    w_ag = nw1_ref[pl.ds(nd, agg.shape[1]), :]
    x = (jnp.dot(nf, w_nf, preferred_element_type=jnp.float32)
         + jnp.dot(agg, w_ag, preferred_element_type=jnp.float32)
         + nb1_ref[...])
    h = _gelu(x)
    upd = jnp.dot(h, nw2_ref[...], preferred_element_type=jnp.float32)
    no = nf + upd + nb2_ref[...]
    mu = jnp.mean(no, axis=-1, keepdims=True)
    var = jnp.mean((no - mu) ** 2, axis=-1, keepdims=True)
    no_ref[...] = (no - mu) * lax.rsqrt(var + 1e-5) * ng_ref[...] + nbn_ref[...]


def _make_sc_gather(E, ND, NC, NS, K):
    NW = NC * NS
    EW = E // NW
    ITERS = EW // K

    mesh = plsc.VectorSubcoreMesh(core_axis_name="c", subcore_axis_name="s")

    @functools.partial(
        pl.kernel,
        out_type=(jax.ShapeDtypeStruct((E, ND), jnp.float32),
                  jax.ShapeDtypeStruct((E, ND), jnp.float32)),
        mesh=mesh,
        scratch_types=[
            pltpu.VMEM((K,), jnp.int32),
            pltpu.VMEM((K,), jnp.int32),
            pltpu.VMEM((K, ND), jnp.float32),
            pltpu.VMEM((K, ND), jnp.float32),
            pltpu.SemaphoreType.DMA,
            pltpu.SemaphoreType.DMA,
        ],
    )
    def sc_gather(ps_hbm, pd_hbm, src_hbm, dst_hbm, gs_hbm, gd_hbm,
                  idx_s, idx_d, rows_s, rows_d, sem_s, sem_d):
        wid = lax.axis_index("s") * NC + lax.axis_index("c")
        base = wid * EW

        def body(i, _):
            off = base + i * K
            pltpu.sync_copy(src_hbm.at[pl.ds(off, K)], idx_s)
            pltpu.sync_copy(dst_hbm.at[pl.ds(off, K)], idx_d)
            cs = pltpu.async_copy(ps_hbm.at[idx_s], rows_s, sem_s)
            cd = pltpu.async_copy(pd_hbm.at[idx_d], rows_d, sem_d)
            cs.wait()
            cd.wait()
            pltpu.sync_copy(rows_s, gs_hbm.at[pl.ds(off, K)])
            pltpu.sync_copy(rows_d, gd_hbm.at[pl.ds(off, K)])
            return 0

        lax.fori_loop(0, ITERS, body, 0)

    return sc_gather


def _make_sc_scatter(E, N, ED, NC, NS, K):
    EW = E // NS
    ITERS = EW // K
    HALF = N // NC
    CH = 40
    NCHUNK = HALF // CH
    ACC = HALF + CH

    mesh = plsc.VectorSubcoreMesh(core_axis_name="c", subcore_axis_name="s")

    @functools.partial(
        pl.kernel,
        out_type=(jax.ShapeDtypeStruct((N, ED), jnp.float32),
                  jax.ShapeDtypeStruct((N, ED), jnp.float32)),
        mesh=mesh,
        scratch_types=[
            pltpu.VMEM((K,), jnp.int32),
            pltpu.VMEM((K,), jnp.int32),
            pltpu.VMEM((K, ED), jnp.float32),
            pltpu.VMEM((K, ED), jnp.float32),
            pltpu.VMEM((CH, ED), jnp.float32),
            pltpu.VMEM_SHARED((ACC, ED), jnp.float32),
            pltpu.VMEM_SHARED((ACC, ED), jnp.float32),
        ],
    )
    def sc_scatter(eo_hbm, dst_hbm, agg_hbm, deg_hbm,
                   idx_v, idx2_v, eo_v, ones_v, tmp_v, agg_sh, deg_sh):
        cid = lax.axis_index("c")
        sid = lax.axis_index("s")
        base = sid * EW
        lo = cid * HALF

        def fill_ones(i, _):
            ones_v[i, :] = jnp.full((ED,), 1.0, jnp.float32)
            return 0
        lax.fori_loop(0, K, fill_ones, 0)

        def fill_zero(i, _):
            tmp_v[i, :] = jnp.zeros((ED,), jnp.float32)
            return 0
        lax.fori_loop(0, CH, fill_zero, 0)

        def init_chunk(j, _):
            c = sid + j * NS

            @pl.when(c < NCHUNK)
            def _():
                pltpu.sync_copy(tmp_v, agg_sh.at[pl.ds(c * CH, CH)])
                pltpu.sync_copy(tmp_v, deg_sh.at[pl.ds(c * CH, CH)])
            return 0
        lax.fori_loop(0, (NCHUNK + NS - 1) // NS, init_chunk, 0)
        plsc.subcore_barrier()

        def body(i, _):
            off = base + i * K
            pltpu.sync_copy(dst_hbm.at[pl.ds(off, K)], idx_v)
            pltpu.sync_copy(eo_hbm.at[pl.ds(off, K)], eo_v)

            def remap(j, _):
                v = idx_v[pl.ds(j * 16, 16)] - lo
                ok = (v >= 0) & (v < HALF)
                idx2_v[pl.ds(j * 16, 16)] = jnp.where(ok, v, HALF)
                return 0
            lax.fori_loop(0, K // 16, remap, 0)

            pltpu.sync_copy(eo_v, agg_sh.at[idx2_v], add=True)
            pltpu.sync_copy(ones_v, deg_sh.at[idx2_v], add=True)
            return 0

        lax.fori_loop(0, ITERS, body, 0)
        plsc.subcore_barrier()

        def exp_chunk(j, _):
            c = sid + j * NS

            @pl.when(c < NCHUNK)
            def _():
                pltpu.sync_copy(agg_sh.at[pl.ds(c * CH, CH)], tmp_v)
                pltpu.sync_copy(tmp_v, agg_hbm.at[pl.ds(lo + c * CH, CH)])
                pltpu.sync_copy(deg_sh.at[pl.ds(c * CH, CH)], tmp_v)
                pltpu.sync_copy(tmp_v, deg_hbm.at[pl.ds(lo + c * CH, CH)])
            return 0
        lax.fori_loop(0, (NCHUNK + NS - 1) // NS, exp_chunk, 0)

    return sc_scatter


def kernel(node_features, edge_index, edge_features, ew1, eb1, ew2, eb2,
           nw1, nb1, nw2, nb2, eg, ebn, ng, nbn):
    N, ND = node_features.shape
    E, ED = edge_features.shape
    H = ew1.shape[1]
    NC, NS = 2, 16

    src = edge_index[0]
    dst = edge_index[1]

    ps, pd = pl.pallas_call(
        _pre_body,
        out_shape=(jax.ShapeDtypeStruct((N, H), jnp.float32),
                   jax.ShapeDtypeStruct((N, H), jnp.float32)),
    )(node_features, ew1)

    gs, gd = _make_sc_gather(E, H, NC, NS, K=80)(ps, pd, src, dst)

    EB = 2560
    grid = (E // EB,)
    eo = pl.pallas_call(
        _edge_body,
        grid=grid,
        in_specs=[
            pl.BlockSpec((EB, H), lambda i: (i, 0)),
            pl.BlockSpec((EB, H), lambda i: (i, 0)),
            pl.BlockSpec((EB, ED), lambda i: (i, 0)),
            pl.BlockSpec((2 * ND + ED, H), lambda i: (0, 0)),
            pl.BlockSpec((1, H), lambda i: (0, 0)),
            pl.BlockSpec((H, ED), lambda i: (0, 0)),
            pl.BlockSpec((1, ED), lambda i: (0, 0)),
            pl.BlockSpec((1, ED), lambda i: (0, 0)),
            pl.BlockSpec((1, ED), lambda i: (0, 0)),
        ],
        out_specs=pl.BlockSpec((EB, ED), lambda i: (i, 0)),
        out_shape=jax.ShapeDtypeStruct((E, ED), jnp.float32),
    )(gs, gd, edge_features, ew1, eb1.reshape(1, H), ew2,
      eb2.reshape(1, ED), eg.reshape(1, ED), ebn.reshape(1, ED))

    agg, deg = _make_sc_scatter(E, N, ED, NC, NS, K=80)(eo, dst)

    no = pl.pallas_call(
        _node_body,
        out_shape=jax.ShapeDtypeStruct((N, ND), jnp.float32),
    )(node_features, agg, deg, nw1,
      nb1.reshape(1, H), nw2, nb2.reshape(1, ND), ng.reshape(1, ND),
      nbn.reshape(1, ND))

    return (no, eo)


# speedup vs baseline: 4.1567x; 1.4106x over previous
import functools

import jax
import jax.numpy as jnp
from jax import lax
from jax.experimental import pallas as pl
from jax.experimental.pallas import tpu as pltpu
from jax.experimental.pallas import tpu_sc as plsc


def _pre_body(nf_ref, ew1_ref, ps_ref, pd_ref):
    nf = nf_ref[...]
    nd = nf.shape[1]
    ws = ew1_ref[pl.ds(0, nd), :]
    wd = ew1_ref[pl.ds(nd, nd), :]
    ps_ref[...] = jnp.dot(nf, ws, preferred_element_type=jnp.float32)
    pd_ref[...] = jnp.dot(nf, wd, preferred_element_type=jnp.float32)


def _gelu(x):
    return 0.5 * x * (1.0 + lax.erf(x * 0.7071067811865476))


def _edge_body(s_ref, ef_ref, ew1_ref, eb1_ref, ew2_ref, eb2_ref,
               eg_ref, ebn_ref, eo_ref):
    ef = ef_ref[...]
    ed = ef.shape[1]
    nd2 = ew1_ref.shape[0] - ed
    we = ew1_ref[pl.ds(nd2, ed), :]
    x = s_ref[...]
    x = x + jnp.dot(ef, we, preferred_element_type=jnp.float32) + eb1_ref[...]
    h = _gelu(x)
    upd = jnp.dot(h, ew2_ref[...], preferred_element_type=jnp.float32)
    eo = ef + upd + eb2_ref[...]
    mu = jnp.mean(eo, axis=-1, keepdims=True)
    var = jnp.mean((eo - mu) ** 2, axis=-1, keepdims=True)
    eo_ref[...] = (eo - mu) * lax.rsqrt(var + 1e-5) * eg_ref[...] + ebn_ref[...]


def _node_body(nf_ref, agg_ref, deg_ref, nw1_ref,
               nb1_ref, nw2_ref, nb2_ref, ng_ref, nbn_ref, no_ref):
    nf = nf_ref[...]
    nd = nf.shape[1]
    cnt = jnp.maximum(deg_ref[:, 0:1], 1.0)
    agg = agg_ref[...] / cnt
    w_nf = nw1_ref[pl.ds(0, nd), :]
    w_ag = nw1_ref[pl.ds(nd, agg.shape[1]), :]
    x = (jnp.dot(nf, w_nf, preferred_element_type=jnp.float32)
         + jnp.dot(agg, w_ag, preferred_element_type=jnp.float32)
         + nb1_ref[...])
    h = _gelu(x)
    upd = jnp.dot(h, nw2_ref[...], preferred_element_type=jnp.float32)
    no = nf + upd + nb2_ref[...]
    mu = jnp.mean(no, axis=-1, keepdims=True)
    var = jnp.mean((no - mu) ** 2, axis=-1, keepdims=True)
    no_ref[...] = (no - mu) * lax.rsqrt(var + 1e-5) * ng_ref[...] + nbn_ref[...]


def _make_sc_gather(E, ND, NC, NS, K, GBUF=4):
    NW = NC * NS
    EW = E // NW
    CHUNKS = EW // K
    MAIN = (CHUNKS // GBUF) * GBUF

    mesh = plsc.VectorSubcoreMesh(core_axis_name="c", subcore_axis_name="s")

    @functools.partial(
        pl.kernel,
        out_type=jax.ShapeDtypeStruct((E, ND), jnp.float32),
        mesh=mesh,
        scratch_types=(
            [pltpu.VMEM((EW,), jnp.int32), pltpu.VMEM((EW,), jnp.int32)]
            + [pltpu.VMEM((K, ND), jnp.float32) for _ in range(2 * GBUF)]
            + [pltpu.SemaphoreType.DMA for _ in range(GBUF)]
        ),
    )
    def sc_gather(ps_hbm, pd_hbm, src_hbm, dst_hbm, s_hbm,
                  idx_all_s, idx_all_d, *bufs):
        rows_s = bufs[:GBUF]
        rows_d = bufs[GBUF:2 * GBUF]
        sems = bufs[2 * GBUF:]
        wid = lax.axis_index("s") * NC + lax.axis_index("c")
        base = wid * EW

        pltpu.sync_copy(src_hbm.at[pl.ds(base, EW)], idx_all_s)
        pltpu.sync_copy(dst_hbm.at[pl.ds(base, EW)], idx_all_d)

        def add_and_flush(j, off):
            def add_row(r, _):
                for cc in range(ND // 16):
                    sl = pl.ds(cc * 16, 16)
                    rows_s[j][r, sl] = rows_s[j][r, sl] + rows_d[j][r, sl]
                return 0
            lax.fori_loop(0, K, add_row, 0)
            pltpu.sync_copy(rows_s[j], s_hbm.at[pl.ds(base + off, K)])

        def group(g, _):
            descs = []
            for j in range(GBUF):
                off = (g * GBUF + j) * K
                cs = pltpu.async_copy(
                    ps_hbm.at[idx_all_s.at[pl.ds(off, K)]], rows_s[j], sems[j])
                cd = pltpu.async_copy(
                    pd_hbm.at[idx_all_d.at[pl.ds(off, K)]], rows_d[j], sems[j])
                descs.append((cs, cd, off))
            for j, (cs, cd, off) in enumerate(descs):
                cs.wait()
                cd.wait()
                add_and_flush(j, off)
            return 0

        lax.fori_loop(0, MAIN // GBUF, group, 0)

        for c in range(MAIN, CHUNKS):
            off = c * K
            cs = pltpu.async_copy(
                ps_hbm.at[idx_all_s.at[pl.ds(off, K)]], rows_s[0], sems[0])
            cd = pltpu.async_copy(
                pd_hbm.at[idx_all_d.at[pl.ds(off, K)]], rows_d[0], sems[0])
            cs.wait()
            cd.wait()
            add_and_flush(0, off)

    return sc_gather


def _make_sc_scatter(E, N, ED, NC, NS, K):
    EW = E // NS
    CHUNKS = EW // K
    PBUF = 3
    MAIN = (CHUNKS // PBUF) * PBUF
    HALF = N // NC
    CH = 40
    NCHUNK = HALF // CH
    ACC = HALF + 8

    mesh = plsc.VectorSubcoreMesh(core_axis_name="c", subcore_axis_name="s")

    @functools.partial(
        pl.kernel,
        out_type=(jax.ShapeDtypeStruct((N, ED), jnp.float32),
                  jax.ShapeDtypeStruct((N, ED), jnp.float32)),
        mesh=mesh,
        scratch_types=(
            [pltpu.VMEM((K,), jnp.int32) for _ in range(PBUF)]
            + [pltpu.VMEM((K,), jnp.int32) for _ in range(PBUF)]
            + [pltpu.VMEM((K, ED), jnp.float32) for _ in range(PBUF)]
            + [pltpu.SemaphoreType.DMA for _ in range(PBUF)]
            + [pltpu.SemaphoreType.DMA for _ in range(PBUF)]
            + [
                pltpu.VMEM((K, ED), jnp.float32),
                pltpu.VMEM((CH, ED), jnp.float32),
                pltpu.VMEM_SHARED((ACC, ED), jnp.float32),
                pltpu.VMEM_SHARED((ACC, ED), jnp.float32),
            ]
        ),
    )
    def sc_scatter(eo_hbm, dst_hbm, agg_hbm, deg_hbm, *refs):
        idx_v = refs[:PBUF]
        idx2_v = refs[PBUF:2 * PBUF]
        eo_v = refs[2 * PBUF:3 * PBUF]
        sem_l = refs[3 * PBUF:4 * PBUF]
        sem_s = refs[4 * PBUF:5 * PBUF]
        ones_v, tmp_v, agg_sh, deg_sh = refs[5 * PBUF:]
        cid = lax.axis_index("c")
        sid = lax.axis_index("s")
        base = sid * EW
        lo = cid * HALF

        def fill_ones(i, _):
            ones_v[i, :] = jnp.full((ED,), 1.0, jnp.float32)
            return 0
        lax.fori_loop(0, K, fill_ones, 0)

        def fill_zero(i, _):
            tmp_v[i, :] = jnp.zeros((ED,), jnp.float32)
            return 0
        lax.fori_loop(0, CH, fill_zero, 0)

        def init_chunk(j, _):
            c = sid + j * NS

            @pl.when(c < NCHUNK)
            def _():
                pltpu.sync_copy(tmp_v, agg_sh.at[pl.ds(c * CH, CH)])
                pltpu.sync_copy(tmp_v, deg_sh.at[pl.ds(c * CH, CH)])
            return 0
        lax.fori_loop(0, (NCHUNK + NS - 1) // NS, init_chunk, 0)
        plsc.subcore_barrier()

        def remap(b):
            def rm(t, _):
                v = idx_v[b][pl.ds(t * 16, 16)] - lo
                ok = (v >= 0) & (v < HALF)
                idx2_v[b][pl.ds(t * 16, 16)] = jnp.where(ok, v, HALF)
                return 0
            lax.fori_loop(0, K // 16, rm, 0)

        def do_chunks(first, count):
            loads = []
            for b in range(count):
                off = base + (first + b) * K
                li = pltpu.async_copy(dst_hbm.at[pl.ds(off, K)],
                                      idx_v[b], sem_l[b])
                le = pltpu.async_copy(eo_hbm.at[pl.ds(off, K)],
                                      eo_v[b], sem_l[b])
                loads.append((li, le))
            scats = []
            for b, (li, le) in enumerate(loads):
                li.wait()
                le.wait()
                remap(b)
                sa = pltpu.async_copy(eo_v[b], agg_sh.at[idx2_v[b]],
                                      sem_s[b], add=True)
                sd = pltpu.async_copy(ones_v, deg_sh.at[idx2_v[b]],
                                      sem_s[b], add=True)
                scats.append((sa, sd))
            for sa, sd in scats:
                sa.wait()
                sd.wait()

        def group(g, _):
            do_chunks(g * PBUF, PBUF)
            return 0

        lax.fori_loop(0, MAIN // PBUF, group, 0)
        if MAIN < CHUNKS:
            do_chunks(MAIN, CHUNKS - MAIN)
        plsc.subcore_barrier()

        def exp_chunk(j, _):
            c = sid + j * NS

            @pl.when(c < NCHUNK)
            def _():
                pltpu.sync_copy(agg_sh.at[pl.ds(c * CH, CH)], tmp_v)
                pltpu.sync_copy(tmp_v, agg_hbm.at[pl.ds(lo + c * CH, CH)])
                pltpu.sync_copy(deg_sh.at[pl.ds(c * CH, CH)], tmp_v)
                pltpu.sync_copy(tmp_v, deg_hbm.at[pl.ds(lo + c * CH, CH)])
            return 0
        lax.fori_loop(0, (NCHUNK + NS - 1) // NS, exp_chunk, 0)

    return sc_scatter


def kernel(node_features, edge_index, edge_features, ew1, eb1, ew2, eb2,
           nw1, nb1, nw2, nb2, eg, ebn, ng, nbn):
    N, ND = node_features.shape
    E, ED = edge_features.shape
    H = ew1.shape[1]
    NC, NS = 2, 16

    src = edge_index[0]
    dst = edge_index[1]

    ps, pd = pl.pallas_call(
        _pre_body,
        out_shape=(jax.ShapeDtypeStruct((N, H), jnp.float32),
                   jax.ShapeDtypeStruct((N, H), jnp.float32)),
    )(node_features, ew1)

    s = _make_sc_gather(E, H, NC, NS, K=80)(ps, pd, src, dst)

    EB = 2560
    grid = (E // EB,)
    eo = pl.pallas_call(
        _edge_body,
        grid=grid,
        in_specs=[
            pl.BlockSpec((EB, H), lambda i: (i, 0)),
            pl.BlockSpec((EB, ED), lambda i: (i, 0)),
            pl.BlockSpec((2 * ND + ED, H), lambda i: (0, 0)),
            pl.BlockSpec((1, H), lambda i: (0, 0)),
            pl.BlockSpec((H, ED), lambda i: (0, 0)),
            pl.BlockSpec((1, ED), lambda i: (0, 0)),
            pl.BlockSpec((1, ED), lambda i: (0, 0)),
            pl.BlockSpec((1, ED), lambda i: (0, 0)),
        ],
        out_specs=pl.BlockSpec((EB, ED), lambda i: (i, 0)),
        out_shape=jax.ShapeDtypeStruct((E, ED), jnp.float32),
    )(s, edge_features, ew1, eb1.reshape(1, H), ew2,
      eb2.reshape(1, ED), eg.reshape(1, ED), ebn.reshape(1, ED))

    agg, deg = _make_sc_scatter(E, N, ED, NC, NS, K=80)(eo, dst)

    no = pl.pallas_call(
        _node_body,
        out_shape=jax.ShapeDtypeStruct((N, ND), jnp.float32),
    )(node_features, agg, deg, nw1,
      nb1.reshape(1, H), nw2, nb2.reshape(1, ND), ng.reshape(1, ND),
      nbn.reshape(1, ND))

    return (no, eo)


# speedup vs baseline: 4.5611x; 1.0973x over previous
import functools

import jax
import jax.numpy as jnp
from jax import lax
from jax.experimental import pallas as pl
from jax.experimental.pallas import tpu as pltpu
from jax.experimental.pallas import tpu_sc as plsc


def _pre_body(nf_ref, ew1_ref, ps_ref, pd_ref):
    nf = nf_ref[...]
    nd = nf.shape[1]
    ws = ew1_ref[pl.ds(0, nd), :]
    wd = ew1_ref[pl.ds(nd, nd), :]
    ps_ref[...] = jnp.dot(nf, ws, preferred_element_type=jnp.float32)
    pd_ref[...] = jnp.dot(nf, wd, preferred_element_type=jnp.float32)


def _gelu(x):
    return 0.5 * x * (1.0 + lax.erf(x * 0.7071067811865476))


def _edge_body(s_ref, ef_ref, ew1_ref, eb1_ref, ew2_ref, eb2_ref,
               eg_ref, ebn_ref, eo_ref, pay_ref):
    ef = ef_ref[...]
    ed = ef.shape[1]
    nd2 = ew1_ref.shape[0] - ed
    we = ew1_ref[pl.ds(nd2, ed), :]
    x = s_ref[...]
    x = x + jnp.dot(ef, we, preferred_element_type=jnp.float32) + eb1_ref[...]
    h = _gelu(x)
    upd = jnp.dot(h, ew2_ref[...], preferred_element_type=jnp.float32)
    eo = ef + upd + eb2_ref[...]
    mu = jnp.mean(eo, axis=-1, keepdims=True)
    var = jnp.mean((eo - mu) ** 2, axis=-1, keepdims=True)
    eo = (eo - mu) * lax.rsqrt(var + 1e-5) * eg_ref[...] + ebn_ref[...]
    eo_ref[...] = eo
    pay_ref[...] = jnp.concatenate(
        [eo, jnp.ones(eo.shape, jnp.float32)], axis=-1)


def _node_body(nf_ref, acc_ref, nw1_ref,
               nb1_ref, nw2_ref, nb2_ref, ng_ref, nbn_ref, no_ref):
    nf = nf_ref[...]
    nd = nf.shape[1]
    ed = acc_ref.shape[1] // 2
    cnt = jnp.maximum(acc_ref[:, ed:ed + 1], 1.0)
    agg = acc_ref[:, 0:ed] / cnt
    w_nf = nw1_ref[pl.ds(0, nd), :]
    w_ag = nw1_ref[pl.ds(nd, agg.shape[1]), :]
    x = (jnp.dot(nf, w_nf, preferred_element_type=jnp.float32)
         + jnp.dot(agg, w_ag, preferred_element_type=jnp.float32)
         + nb1_ref[...])
    h = _gelu(x)
    upd = jnp.dot(h, nw2_ref[...], preferred_element_type=jnp.float32)
    no = nf + upd + nb2_ref[...]
    mu = jnp.mean(no, axis=-1, keepdims=True)
    var = jnp.mean((no - mu) ** 2, axis=-1, keepdims=True)
    no_ref[...] = (no - mu) * lax.rsqrt(var + 1e-5) * ng_ref[...] + nbn_ref[...]


def _make_sc_gather(E, ND, NC, NS, K, GBUF=3):
    NW = NC * NS
    EW = E // NW
    FULL = EW // K
    MAIN = (FULL // GBUF) * GBUF
    CHUNKS = FULL + (1 if EW % K else 0)

    mesh = plsc.VectorSubcoreMesh(core_axis_name="c", subcore_axis_name="s")

    @functools.partial(
        pl.kernel,
        out_type=jax.ShapeDtypeStruct((E, ND), jnp.float32),
        mesh=mesh,
        scratch_types=(
            [pltpu.VMEM((EW,), jnp.int32), pltpu.VMEM((EW,), jnp.int32)]
            + [pltpu.VMEM((K, ND), jnp.float32) for _ in range(2 * GBUF)]
            + [pltpu.SemaphoreType.DMA for _ in range(2 * GBUF)]
        ),
    )
    def sc_gather(ps_hbm, pd_hbm, src_hbm, dst_hbm, s_hbm,
                  idx_all_s, idx_all_d, *bufs):
        rows_s = bufs[:GBUF]
        rows_d = bufs[GBUF:2 * GBUF]
        sem_g = bufs[2 * GBUF:3 * GBUF]
        sem_w = bufs[3 * GBUF:]
        wid = lax.axis_index("s") * NC + lax.axis_index("c")
        base = wid * EW

        pltpu.sync_copy(src_hbm.at[pl.ds(base, EW)], idx_all_s)
        pltpu.sync_copy(dst_hbm.at[pl.ds(base, EW)], idx_all_d)

        def add_rows(j, n):
            def add_row(r, _):
                for cc in range(ND // 16):
                    sl = pl.ds(cc * 16, 16)
                    rows_s[j][r, sl] = rows_s[j][r, sl] + rows_d[j][r, sl]
                return 0
            lax.fori_loop(0, n, add_row, 0)

        def drain_w(j):
            pltpu.make_async_copy(
                rows_s[j], s_hbm.at[pl.ds(base, K)], sem_w[j]).wait()

        def group(g, _):
            descs = []
            for j in range(GBUF):
                off = (g * GBUF + j) * K

                @pl.when(g > 0)
                def _():
                    drain_w(j)
                cs = pltpu.async_copy(
                    ps_hbm.at[idx_all_s.at[pl.ds(off, K)]], rows_s[j],
                    sem_g[j])
                cd = pltpu.async_copy(
                    pd_hbm.at[idx_all_d.at[pl.ds(off, K)]], rows_d[j],
                    sem_g[j])
                descs.append((cs, cd, off))
            for j, (cs, cd, off) in enumerate(descs):
                cs.wait()
                cd.wait()
                add_rows(j, K)
                pltpu.async_copy(rows_s[j], s_hbm.at[pl.ds(base + off, K)],
                                 sem_w[j])
            return 0

        NGROUP = MAIN // GBUF
        lax.fori_loop(0, NGROUP, group, 0)

        done0 = False
        for c in range(MAIN, CHUNKS):
            off = c * K
            kt = min(K, EW - off)
            if NGROUP > 0 and not done0:
                drain_w(0)
                done0 = True
            cs = pltpu.async_copy(
                ps_hbm.at[idx_all_s.at[pl.ds(off, kt)]],
                rows_s[0].at[pl.ds(0, kt)], sem_g[0])
            cd = pltpu.async_copy(
                pd_hbm.at[idx_all_d.at[pl.ds(off, kt)]],
                rows_d[0].at[pl.ds(0, kt)], sem_g[0])
            cs.wait()
            cd.wait()
            add_rows(0, kt)
            pltpu.sync_copy(rows_s[0].at[pl.ds(0, kt)],
                            s_hbm.at[pl.ds(base + off, kt)])

        if NGROUP > 0:
            for j in range(GBUF):
                if j == 0 and done0:
                    continue
                drain_w(j)

    return sc_gather


def _make_sc_scatter(E, N, ED, NC, NS, K, PBUF=4):
    EW = E // NS
    FULL = EW // K
    MAIN = (FULL // PBUF) * PBUF
    TAIL = EW - FULL * K
    HALF = N // NC
    CH = 40
    NCHUNK = HALF // CH
    ACC = HALF + 8
    ED2 = 2 * ED

    mesh = plsc.VectorSubcoreMesh(core_axis_name="c", subcore_axis_name="s")

    @functools.partial(
        pl.kernel,
        out_type=jax.ShapeDtypeStruct((N, ED2), jnp.float32),
        mesh=mesh,
        scratch_types=(
            [pltpu.VMEM((K,), jnp.int32) for _ in range(PBUF)]
            + [pltpu.VMEM((K,), jnp.int32) for _ in range(PBUF)]
            + [pltpu.VMEM((K, ED2), jnp.float32) for _ in range(PBUF)]
            + [pltpu.SemaphoreType.DMA for _ in range(PBUF)]
            + [pltpu.SemaphoreType.DMA for _ in range(PBUF)]
            + [
                pltpu.VMEM((max(TAIL, 8),), jnp.int32),
                pltpu.VMEM((max(TAIL, 8),), jnp.int32),
                pltpu.VMEM((CH, ED2), jnp.float32),
                pltpu.VMEM_SHARED((ACC, ED2), jnp.float32),
            ]
        ),
    )
    def sc_scatter(pay_hbm, dst_hbm, acc_hbm, *refs):
        idx_v = refs[:PBUF]
        idx2_v = refs[PBUF:2 * PBUF]
        pay_v = refs[2 * PBUF:3 * PBUF]
        sem_l = refs[3 * PBUF:4 * PBUF]
        sem_s = refs[4 * PBUF:5 * PBUF]
        idx_t, idx2_t, tmp_v, acc_sh = refs[5 * PBUF:]
        cid = lax.axis_index("c")
        sid = lax.axis_index("s")
        base = sid * EW
        lo = cid * HALF

        def fill_zero(i, _):
            tmp_v[i, :] = jnp.zeros((ED2,), jnp.float32)
            return 0
        lax.fori_loop(0, CH, fill_zero, 0)

        def init_chunk(j, _):
            c = sid + j * NS

            @pl.when(c < NCHUNK)
            def _():
                pltpu.sync_copy(tmp_v, acc_sh.at[pl.ds(c * CH, CH)])
            return 0
        lax.fori_loop(0, (NCHUNK + NS - 1) // NS, init_chunk, 0)
        plsc.subcore_barrier()

        def remap(src_idx, dst_idx, n):
            def rm(t, _):
                v = src_idx[pl.ds(t * 16, 16)] - lo
                ok = (v >= 0) & (v < HALF)
                dst_idx[pl.ds(t * 16, 16)] = jnp.where(ok, v, HALF)
                return 0
            lax.fori_loop(0, n // 16, rm, 0)

        def drain_scat(b):
            pltpu.make_async_copy(
                pay_v[b], acc_sh.at[idx2_v[b]], sem_s[b]).wait()

        def group(g, _):
            loads = []
            for b in range(PBUF):
                off = base + (g * PBUF + b) * K

                @pl.when(g > 0)
                def _():
                    drain_scat(b)
                li = pltpu.async_copy(dst_hbm.at[pl.ds(off, K)],
                                      idx_v[b], sem_l[b])
                lp = pltpu.async_copy(pay_hbm.at[pl.ds(off, K)],
                                      pay_v[b], sem_l[b])
                loads.append((li, lp))
            for b, (li, lp) in enumerate(loads):
                li.wait()
                lp.wait()
                remap(idx_v[b], idx2_v[b], K)
                pltpu.async_copy(pay_v[b], acc_sh.at[idx2_v[b]],
                                 sem_s[b], add=True)
            return 0

        NGROUP = MAIN // PBUF
        lax.fori_loop(0, NGROUP, group, 0)

        for i, c in enumerate(range(MAIN, FULL)):
            b = i % PBUF
            off = base + c * K
            if NGROUP > 0 or i >= PBUF:
                drain_scat(b)
            li = pltpu.async_copy(dst_hbm.at[pl.ds(off, K)], idx_v[b],
                                  sem_l[b])
            lp = pltpu.async_copy(pay_hbm.at[pl.ds(off, K)], pay_v[b],
                                  sem_l[b])
            li.wait()
            lp.wait()
            remap(idx_v[b], idx2_v[b], K)
            pltpu.async_copy(pay_v[b], acc_sh.at[idx2_v[b]], sem_s[b],
                             add=True)

        if TAIL:
            off = base + FULL * K
            b = (FULL - MAIN) % PBUF
            if FULL > 0:
                drain_scat(b)
            li = pltpu.async_copy(dst_hbm.at[pl.ds(off, TAIL)], idx_t,
                                  sem_l[b])
            lp = pltpu.async_copy(pay_hbm.at[pl.ds(off, TAIL)],
                                  pay_v[b].at[pl.ds(0, TAIL)], sem_l[b])
            li.wait()
            lp.wait()
            remap(idx_t, idx2_t, TAIL)
            pltpu.async_copy(pay_v[b].at[pl.ds(0, TAIL)],
                             acc_sh.at[idx2_t], sem_s[b], add=True).wait()

        nleft = FULL - MAIN
        for b in range(PBUF):
            used_main = NGROUP > 0 or b < nleft
            tail_b = TAIL and b == (FULL - MAIN) % PBUF
            if used_main and not tail_b:
                drain_scat(b)
        plsc.subcore_barrier()

        def exp_chunk(j, _):
            c = sid + j * NS

            @pl.when(c < NCHUNK)
            def _():
                pltpu.sync_copy(acc_sh.at[pl.ds(c * CH, CH)], tmp_v)
                pltpu.sync_copy(tmp_v, acc_hbm.at[pl.ds(lo + c * CH, CH)])
            return 0
        lax.fori_loop(0, (NCHUNK + NS - 1) // NS, exp_chunk, 0)

    return sc_scatter


def kernel(node_features, edge_index, edge_features, ew1, eb1, ew2, eb2,
           nw1, nb1, nw2, nb2, eg, ebn, ng, nbn):
    N, ND = node_features.shape
    E, ED = edge_features.shape
    H = ew1.shape[1]
    NC, NS = 2, 16

    src = edge_index[0]
    dst = edge_index[1]

    ps, pd = pl.pallas_call(
        _pre_body,
        out_shape=(jax.ShapeDtypeStruct((N, H), jnp.float32),
                   jax.ShapeDtypeStruct((N, H), jnp.float32)),
    )(node_features, ew1)

    s = _make_sc_gather(E, H, NC, NS, K=128)(ps, pd, src, dst)

    EB = 2560
    grid = (E // EB,)
    eo, pay = pl.pallas_call(
        _edge_body,
        grid=grid,
        in_specs=[
            pl.BlockSpec((EB, H), lambda i: (i, 0)),
            pl.BlockSpec((EB, ED), lambda i: (i, 0)),
            pl.BlockSpec((2 * ND + ED, H), lambda i: (0, 0)),
            pl.BlockSpec((1, H), lambda i: (0, 0)),
            pl.BlockSpec((H, ED), lambda i: (0, 0)),
            pl.BlockSpec((1, ED), lambda i: (0, 0)),
            pl.BlockSpec((1, ED), lambda i: (0, 0)),
            pl.BlockSpec((1, ED), lambda i: (0, 0)),
        ],
        out_specs=(pl.BlockSpec((EB, ED), lambda i: (i, 0)),
                   pl.BlockSpec((EB, 2 * ED), lambda i: (i, 0))),
        out_shape=(jax.ShapeDtypeStruct((E, ED), jnp.float32),
                   jax.ShapeDtypeStruct((E, 2 * ED), jnp.float32)),
    )(s, edge_features, ew1, eb1.reshape(1, H), ew2,
      eb2.reshape(1, ED), eg.reshape(1, ED), ebn.reshape(1, ED))

    acc = _make_sc_scatter(E, N, ED, NC, NS, K=128)(pay, dst)

    no = pl.pallas_call(
        _node_body,
        out_shape=jax.ShapeDtypeStruct((N, ND), jnp.float32),
    )(node_features, acc, nw1,
      nb1.reshape(1, H), nw2, nb2.reshape(1, ND), ng.reshape(1, ND),
      nbn.reshape(1, ND))

    return (no, eo)


# speedup vs baseline: 4.8262x; 1.0581x over previous
import functools

import jax
import jax.numpy as jnp
from jax import lax
from jax.experimental import pallas as pl
from jax.experimental.pallas import tpu as pltpu
from jax.experimental.pallas import tpu_sc as plsc


def _pre_body(nf_ref, ew1_ref, ps_ref, pd_ref):
    nf = nf_ref[...]
    nd = nf.shape[1]
    ws = ew1_ref[pl.ds(0, nd), :]
    wd = ew1_ref[pl.ds(nd, nd), :]
    ps_ref[...] = jnp.dot(nf, ws, preferred_element_type=jnp.float32)
    pd_ref[...] = jnp.dot(nf, wd, preferred_element_type=jnp.float32)


def _gelu(x):
    return 0.5 * x * (1.0 + lax.erf(x * 0.7071067811865476))


def _edge_body(s_ref, ef_ref, ew1_ref, eb1_ref, ew2_ref, eb2_ref,
               eg_ref, ebn_ref, eo_ref, pay_ref):
    ef = ef_ref[...]
    ed = ef.shape[1]
    nd2 = ew1_ref.shape[0] - ed
    we = ew1_ref[pl.ds(nd2, ed), :]
    x = s_ref[...]
    x = x + jnp.dot(ef, we, preferred_element_type=jnp.float32) + eb1_ref[...]
    h = _gelu(x)
    upd = jnp.dot(h, ew2_ref[...], preferred_element_type=jnp.float32)
    eo = ef + upd + eb2_ref[...]
    mu = jnp.mean(eo, axis=-1, keepdims=True)
    var = jnp.mean((eo - mu) ** 2, axis=-1, keepdims=True)
    eo = (eo - mu) * lax.rsqrt(var + 1e-5) * eg_ref[...] + ebn_ref[...]
    eo_ref[...] = eo
    pay_ref[...] = jnp.concatenate(
        [eo, jnp.ones(eo.shape, jnp.float32)], axis=-1)


def _node_body(nf_ref, acc_ref, nw1_ref,
               nb1_ref, nw2_ref, nb2_ref, ng_ref, nbn_ref, no_ref):
    nf = nf_ref[...]
    nd = nf.shape[1]
    ed = acc_ref.shape[1] // 2
    cnt = jnp.maximum(acc_ref[:, ed:ed + 1], 1.0)
    agg = acc_ref[:, 0:ed] / cnt
    w_nf = nw1_ref[pl.ds(0, nd), :]
    w_ag = nw1_ref[pl.ds(nd, agg.shape[1]), :]
    x = (jnp.dot(nf, w_nf, preferred_element_type=jnp.float32)
         + jnp.dot(agg, w_ag, preferred_element_type=jnp.float32)
         + nb1_ref[...])
    h = _gelu(x)
    upd = jnp.dot(h, nw2_ref[...], preferred_element_type=jnp.float32)
    no = nf + upd + nb2_ref[...]
    mu = jnp.mean(no, axis=-1, keepdims=True)
    var = jnp.mean((no - mu) ** 2, axis=-1, keepdims=True)
    no_ref[...] = (no - mu) * lax.rsqrt(var + 1e-5) * ng_ref[...] + nbn_ref[...]


def _make_sc_gather(E, ND, NC, NS, K, GBUF=3):
    NW = NC * NS
    EW = E // NW
    FULL = EW // K
    MAIN = (FULL // GBUF) * GBUF
    CHUNKS = FULL + (1 if EW % K else 0)

    mesh = plsc.VectorSubcoreMesh(core_axis_name="c", subcore_axis_name="s")

    @functools.partial(
        pl.kernel,
        out_type=jax.ShapeDtypeStruct((E, ND), jnp.float32),
        mesh=mesh,
        scratch_types=(
            [pltpu.VMEM((EW,), jnp.int32), pltpu.VMEM((EW,), jnp.int32)]
            + [pltpu.VMEM((K, ND), jnp.float32) for _ in range(2 * GBUF)]
            + [pltpu.SemaphoreType.DMA for _ in range(2 * GBUF)]
        ),
    )
    def sc_gather(ps_hbm, pd_hbm, src_hbm, dst_hbm, s_hbm,
                  idx_all_s, idx_all_d, *bufs):
        rows_s = bufs[:GBUF]
        rows_d = bufs[GBUF:2 * GBUF]
        sem_g = bufs[2 * GBUF:3 * GBUF]
        sem_w = bufs[3 * GBUF:]
        wid = lax.axis_index("s") * NC + lax.axis_index("c")
        base = wid * EW

        pltpu.sync_copy(src_hbm.at[pl.ds(base, EW)], idx_all_s)
        pltpu.sync_copy(dst_hbm.at[pl.ds(base, EW)], idx_all_d)

        def add_rows(j, n):
            def add_row(r, _):
                for cc in range(ND // 16):
                    sl = pl.ds(cc * 16, 16)
                    rows_s[j][r, sl] = rows_s[j][r, sl] + rows_d[j][r, sl]
                return 0
            lax.fori_loop(0, n, add_row, 0)

        def drain_w(j):
            pltpu.make_async_copy(
                rows_s[j], s_hbm.at[pl.ds(base, K)], sem_w[j]).wait()

        def group(g, _):
            descs = []
            for j in range(GBUF):
                off = (g * GBUF + j) * K

                @pl.when(g > 0)
                def _():
                    drain_w(j)
                cs = pltpu.async_copy(
                    ps_hbm.at[idx_all_s.at[pl.ds(off, K)]], rows_s[j],
                    sem_g[j])
                cd = pltpu.async_copy(
                    pd_hbm.at[idx_all_d.at[pl.ds(off, K)]], rows_d[j],
                    sem_g[j])
                descs.append((cs, cd, off))
            for j, (cs, cd, off) in enumerate(descs):
                cs.wait()
                cd.wait()
                add_rows(j, K)
                pltpu.async_copy(rows_s[j], s_hbm.at[pl.ds(base + off, K)],
                                 sem_w[j])
            return 0

        NGROUP = MAIN // GBUF
        lax.fori_loop(0, NGROUP, group, 0)

        done0 = False
        for c in range(MAIN, CHUNKS):
            off = c * K
            kt = min(K, EW - off)
            if NGROUP > 0 and not done0:
                drain_w(0)
                done0 = True
            cs = pltpu.async_copy(
                ps_hbm.at[idx_all_s.at[pl.ds(off, kt)]],
                rows_s[0].at[pl.ds(0, kt)], sem_g[0])
            cd = pltpu.async_copy(
                pd_hbm.at[idx_all_d.at[pl.ds(off, kt)]],
                rows_d[0].at[pl.ds(0, kt)], sem_g[0])
            cs.wait()
            cd.wait()
            add_rows(0, kt)
            pltpu.sync_copy(rows_s[0].at[pl.ds(0, kt)],
                            s_hbm.at[pl.ds(base + off, kt)])

        if NGROUP > 0:
            for j in range(GBUF):
                if j == 0 and done0:
                    continue
                drain_w(j)

    return sc_gather


def _make_sc_scatter(E, N, ED, NC, NS, K, PBUF=4):
    EW = E // NS
    FULL = EW // K
    MAIN = (FULL // PBUF) * PBUF
    TAIL = EW - FULL * K
    HALF = N // NC
    CH = 40
    NCHUNK = HALF // CH
    ACC = HALF + 8
    ED2 = 2 * ED

    mesh = plsc.VectorSubcoreMesh(core_axis_name="c", subcore_axis_name="s")

    @functools.partial(
        pl.kernel,
        out_type=jax.ShapeDtypeStruct((N, ED2), jnp.float32),
        mesh=mesh,
        scratch_types=(
            [pltpu.VMEM((K,), jnp.int32) for _ in range(PBUF)]
            + [pltpu.VMEM((K,), jnp.int32) for _ in range(PBUF)]
            + [pltpu.VMEM((K, ED2), jnp.float32) for _ in range(PBUF)]
            + [pltpu.SemaphoreType.DMA for _ in range(PBUF)]
            + [pltpu.SemaphoreType.DMA for _ in range(PBUF)]
            + [
                pltpu.VMEM((max(TAIL, 8),), jnp.int32),
                pltpu.VMEM((max(TAIL, 8),), jnp.int32),
                pltpu.VMEM((CH, ED2), jnp.float32),
                pltpu.VMEM_SHARED((ACC, ED2), jnp.float32),
            ]
        ),
    )
    def sc_scatter(pay_hbm, dst_hbm, acc_hbm, *refs):
        idx_v = refs[:PBUF]
        idx2_v = refs[PBUF:2 * PBUF]
        pay_v = refs[2 * PBUF:3 * PBUF]
        sem_l = refs[3 * PBUF:4 * PBUF]
        sem_s = refs[4 * PBUF:5 * PBUF]
        idx_t, idx2_t, tmp_v, acc_sh = refs[5 * PBUF:]
        cid = lax.axis_index("c")
        sid = lax.axis_index("s")
        base = sid * EW
        lo = cid * HALF

        def fill_zero(i, _):
            tmp_v[i, :] = jnp.zeros((ED2,), jnp.float32)
            return 0
        lax.fori_loop(0, CH, fill_zero, 0)

        def init_chunk(j, _):
            c = sid + j * NS

            @pl.when(c < NCHUNK)
            def _():
                pltpu.sync_copy(tmp_v, acc_sh.at[pl.ds(c * CH, CH)])
            return 0
        lax.fori_loop(0, (NCHUNK + NS - 1) // NS, init_chunk, 0)
        plsc.subcore_barrier()

        def remap(src_idx, dst_idx, n):
            def rm(t, _):
                v = src_idx[pl.ds(t * 16, 16)] - lo
                ok = (v >= 0) & (v < HALF)
                dst_idx[pl.ds(t * 16, 16)] = jnp.where(ok, v, HALF)
                return 0
            lax.fori_loop(0, n // 16, rm, 0)

        def drain_scat(b):
            pltpu.make_async_copy(
                pay_v[b], acc_sh.at[idx2_v[b]], sem_s[b]).wait()

        def group(g, _):
            loads = []
            for b in range(PBUF):
                off = base + (g * PBUF + b) * K

                @pl.when(g > 0)
                def _():
                    drain_scat(b)
                li = pltpu.async_copy(dst_hbm.at[pl.ds(off, K)],
                                      idx_v[b], sem_l[b])
                lp = pltpu.async_copy(pay_hbm.at[pl.ds(off, K)],
                                      pay_v[b], sem_l[b])
                loads.append((li, lp))
            for b, (li, lp) in enumerate(loads):
                li.wait()
                lp.wait()
                remap(idx_v[b], idx2_v[b], K)
                pltpu.async_copy(pay_v[b], acc_sh.at[idx2_v[b]],
                                 sem_s[b], add=True)
            return 0

        NGROUP = MAIN // PBUF
        lax.fori_loop(0, NGROUP, group, 0)

        for i, c in enumerate(range(MAIN, FULL)):
            b = i % PBUF
            off = base + c * K
            if NGROUP > 0 or i >= PBUF:
                drain_scat(b)
            li = pltpu.async_copy(dst_hbm.at[pl.ds(off, K)], idx_v[b],
                                  sem_l[b])
            lp = pltpu.async_copy(pay_hbm.at[pl.ds(off, K)], pay_v[b],
                                  sem_l[b])
            li.wait()
            lp.wait()
            remap(idx_v[b], idx2_v[b], K)
            pltpu.async_copy(pay_v[b], acc_sh.at[idx2_v[b]], sem_s[b],
                             add=True)

        if TAIL:
            off = base + FULL * K
            b = (FULL - MAIN) % PBUF
            if FULL > 0:
                drain_scat(b)
            li = pltpu.async_copy(dst_hbm.at[pl.ds(off, TAIL)], idx_t,
                                  sem_l[b])
            lp = pltpu.async_copy(pay_hbm.at[pl.ds(off, TAIL)],
                                  pay_v[b].at[pl.ds(0, TAIL)], sem_l[b])
            li.wait()
            lp.wait()
            remap(idx_t, idx2_t, TAIL)
            pltpu.async_copy(pay_v[b].at[pl.ds(0, TAIL)],
                             acc_sh.at[idx2_t], sem_s[b], add=True).wait()

        nleft = FULL - MAIN
        for b in range(PBUF):
            used_main = NGROUP > 0 or b < nleft
            tail_b = TAIL and b == (FULL - MAIN) % PBUF
            if used_main and not tail_b:
                drain_scat(b)
        plsc.subcore_barrier()

        def exp_chunk(j, _):
            c = sid + j * NS

            @pl.when(c < NCHUNK)
            def _():
                pltpu.sync_copy(acc_sh.at[pl.ds(c * CH, CH)], tmp_v)
                pltpu.sync_copy(tmp_v, acc_hbm.at[pl.ds(lo + c * CH, CH)])
            return 0
        lax.fori_loop(0, (NCHUNK + NS - 1) // NS, exp_chunk, 0)

    return sc_scatter


def kernel(node_features, edge_index, edge_features, ew1, eb1, ew2, eb2,
           nw1, nb1, nw2, nb2, eg, ebn, ng, nbn):
    N, ND = node_features.shape
    E, ED = edge_features.shape
    H = ew1.shape[1]
    NC, NS = 2, 16

    src = edge_index[0]
    dst = edge_index[1]

    ps, pd = pl.pallas_call(
        _pre_body,
        out_shape=(jax.ShapeDtypeStruct((N, H), jnp.float32),
                   jax.ShapeDtypeStruct((N, H), jnp.float32)),
    )(node_features, ew1)

    s = _make_sc_gather(E, H, NC, NS, K=128)(ps, pd, src, dst)

    EB = 6400
    grid = (E // EB,)
    eo, pay = pl.pallas_call(
        _edge_body,
        grid=grid,
        in_specs=[
            pl.BlockSpec((EB, H), lambda i: (i, 0)),
            pl.BlockSpec((EB, ED), lambda i: (i, 0)),
            pl.BlockSpec((2 * ND + ED, H), lambda i: (0, 0)),
            pl.BlockSpec((1, H), lambda i: (0, 0)),
            pl.BlockSpec((H, ED), lambda i: (0, 0)),
            pl.BlockSpec((1, ED), lambda i: (0, 0)),
            pl.BlockSpec((1, ED), lambda i: (0, 0)),
            pl.BlockSpec((1, ED), lambda i: (0, 0)),
        ],
        out_specs=(pl.BlockSpec((EB, ED), lambda i: (i, 0)),
                   pl.BlockSpec((EB, 2 * ED), lambda i: (i, 0))),
        out_shape=(jax.ShapeDtypeStruct((E, ED), jnp.float32),
                   jax.ShapeDtypeStruct((E, 2 * ED), jnp.float32)),
    )(s, edge_features, ew1, eb1.reshape(1, H), ew2,
      eb2.reshape(1, ED), eg.reshape(1, ED), ebn.reshape(1, ED))

    acc = _make_sc_scatter(E, N, ED, NC, NS, K=128)(pay, dst)

    no = pl.pallas_call(
        _node_body,
        out_shape=jax.ShapeDtypeStruct((N, ND), jnp.float32),
    )(node_features, acc, nw1,
      nb1.reshape(1, H), nw2, nb2.reshape(1, ND), ng.reshape(1, ND),
      nbn.reshape(1, ND))

    return (no, eo)
